# baseline (speedup 1.0000x reference)
import functools

import jax
import jax.numpy as jnp
from jax import lax
from jax.experimental import pallas as pl
from jax.experimental.pallas import tpu as pltpu
from jax.experimental.pallas import tpu_sc as plsc

N_ATOMS = 10000
N_BONDS = 320000
MAX_NB = 32
ATOM_FDIM = 128
BOND_FDIM = 144
HIDDEN = 128
DEPTH = 6
N_MOLS = 200

NA_PAD = 10240
NM_PAD = 256

NC, NS = 2, 16
NW = NC * NS
C = 40
BONDS_PER_W = N_BONDS // NW
G_CHUNKS = BONDS_PER_W // C
BONDS_PER_TILE = N_BONDS // NS
S_CHUNKS = BONDS_PER_TILE // C
AROWS_PER_TILE = NA_PAD // NS

ATOMS_PER_W = NA_PAD // NW
SC_ATOMS = 8
R_CHUNKS = ATOMS_PER_W // SC_ATOMS
RG = SC_ATOMS * MAX_NB // 2

H = HIDDEN
HV = H // 16


def _sc_mesh():
    return plsc.VectorSubcoreMesh(core_axis_name="c", subcore_axis_name="s")


GC = 120
GN = BONDS_PER_W // GC
GT = BONDS_PER_W - GN * GC
SN = BONDS_PER_TILE // GC
ST = BONDS_PER_TILE - SN * GC
ZN = AROWS_PER_TILE // GC
ZT = AROWS_PER_TILE - ZN * GC


def _sc_scatter_body(n_hbm, a2b2_hbm, adump_hbm,
                     six, six2, six_t, bv, bv2, A_sh, sem, sem2):
    cid = lax.axis_index("c")
    sid = lax.axis_index("s")
    wid = sid * NC + cid

    def zrow(i, carry):
        for k in range(HV):
            bv[i, pl.ds(k * 16, 16)] = jnp.zeros((16,), jnp.float32)
        return carry
    lax.fori_loop(0, GC, zrow, None)
    zbase = sid * AROWS_PER_TILE
    for j in range(ZN):
        pltpu.sync_copy(bv, A_sh.at[pl.ds(zbase + j * GC, GC)])
    pltpu.sync_copy(bv.at[pl.ds(0, ZT)],
                    A_sh.at[pl.ds(zbase + ZN * GC, ZT)])
    plsc.subcore_barrier()

    six_ = (six, six2)
    bv_ = (bv, bv2)
    sem_ = (sem, sem2)

    def s_f(c, b):
        off = sid * BONDS_PER_TILE + c * GC
        return ((a2b2_hbm.at[pl.ds(off, GC)], six_[b], sem_[b]),
                (n_hbm.at[pl.ds(off, GC)], bv_[b], sem_[b]))

    def s_start(ds):
        for s, d, sm in ds:
            pltpu.async_copy(s, d, sm)

    def s_wait(ds):
        for s, d, sm in ds:
            pltpu.make_async_copy(s, d, sm).wait()

    s_start(s_f(0, 0))

    def s_pair(jj, carry):
        c0 = 2 * jj
        s_wait(s_f(c0, 0))
        s_start(s_f(c0 + 1, 1))
        pltpu.sync_copy(bv_[0], A_sh.at[six_[0]], add=True)
        s_wait(s_f(c0 + 1, 1))
        s_start(s_f(c0 + 2, 0))
        pltpu.sync_copy(bv_[1], A_sh.at[six_[1]], add=True)
        return carry
    lax.fori_loop(0, SN // 2 - 1, s_pair, None)

    c0 = SN - 2
    s_wait(s_f(c0, 0))
    s_start(s_f(c0 + 1, 1))
    pltpu.sync_copy(bv_[0], A_sh.at[six_[0]], add=True)
    s_wait(s_f(c0 + 1, 1))
    pltpu.sync_copy(bv_[1], A_sh.at[six_[1]], add=True)

    offt = sid * BONDS_PER_TILE + SN * GC
    pltpu.sync_copy(a2b2_hbm.at[pl.ds(offt, ST)], six_t)
    pltpu.sync_copy(n_hbm.at[pl.ds(offt, ST)], bv.at[pl.ds(0, ST)])
    pltpu.sync_copy(bv.at[pl.ds(0, ST)], A_sh.at[six_t], add=True)
    plsc.subcore_barrier()

    zbase = sid * AROWS_PER_TILE
    pltpu.sync_copy(A_sh.at[pl.ds(zbase, AROWS_PER_TILE)],
                    adump_hbm.at[cid].at[pl.ds(zbase, AROWS_PER_TILE)])
    plsc.subcore_barrier()


GC2 = 104
GN2 = 96
GT2 = BONDS_PER_W - GN2 * GC2


def _sc_gather_body(n_hbm, b2a_hbm, b2revb_hbm, adump_hbm, m_hbm,
                    ia0, ia1, ir0, ir1, av0, av1, rv0, rv1,
                    ov0, ov1, isem0, isem1, gsem0, gsem1, osem0, osem1):
    cid = lax.axis_index("c")
    sid = lax.axis_index("s")
    wid = sid * NC + cid
    ia = (ia0, ia1)
    ir = (ir0, ir1)
    av = (av0, av1)
    rv = (rv0, rv1)
    ov = (ov0, ov1)
    isem = (isem0, isem1)
    gsem = (gsem0, gsem1)
    osem = (osem0, osem1)
    A_my = adump_hbm.at[cid]

    def gi(c, b):
        off = wid * BONDS_PER_W + c * GC2
        return ((b2a_hbm.at[pl.ds(off, GC2)], ia[b], isem[b]),
                (b2revb_hbm.at[pl.ds(off, GC2)], ir[b], isem[b]))

    def gg(c, b):
        return ((A_my.at[ia[b]], av[b], gsem[b]),
                (n_hbm.at[ir[b]], rv[b], gsem[b]))

    def go(c, b):
        off = wid * BONDS_PER_W + c * GC2
        return ((ov[b], m_hbm.at[pl.ds(off, GC2)], osem[b]),)

    def start(ds):
        for s, d, sm in ds:
            pltpu.async_copy(s, d, sm)

    def wait(ds):
        for s, d, sm in ds:
            pltpu.make_async_copy(s, d, sm).wait()

    def compute(c, b):
        def row(i, carry):
            for k in range(HV):
                sl = (i, pl.ds(k * 16, 16))
                ov[b][sl] = av[b][sl] - rv[b][sl]
            return carry
        lax.fori_loop(0, GC2, row, None)
        start(go(c, b))

    start(gi(0, 0))
    start(gi(1, 1))
    wait(gi(0, 0))
    start(gg(0, 0))

    wait(gg(0, 0))
    wait(gi(1, 1))
    start(gg(1, 1))
    compute(0, 0)
    start(gi(2, 0))

    wait(gg(1, 1))
    wait(gi(2, 0))
    start(gg(2, 0))
    compute(1, 1)
    start(gi(3, 1))

    def pair(jj, carry):
        c0 = 2 * jj
        wait(gg(c0, 0))
        wait(gi(c0 + 1, 1))
        start(gg(c0 + 1, 1))
        wait(go(c0 - 2, 0))
        compute(c0, 0)
        start(gi(c0 + 2, 0))

        wait(gg(c0 + 1, 1))
        wait(gi(c0 + 2, 0))
        start(gg(c0 + 2, 0))
        wait(go(c0 - 1, 1))
        compute(c0 + 1, 1)
        start(gi(c0 + 3, 1))
        return carry
    lax.fori_loop(1, GN2 // 2 - 1, pair, None)

    ct = GN2 - 2
    wait(gg(ct, 0))
    wait(gi(ct + 1, 1))
    start(gg(ct + 1, 1))
    wait(go(ct - 2, 0))
    compute(ct, 0)

    wait(gg(ct + 1, 1))
    wait(go(ct - 1, 1))
    compute(ct + 1, 1)
    wait(go(ct, 0))
    wait(go(ct + 1, 1))

    offt = wid * BONDS_PER_W + GN2 * GC2
    pltpu.sync_copy(b2a_hbm.at[pl.ds(offt, GT2)], ia0.at[pl.ds(0, GT2)])
    pltpu.sync_copy(b2revb_hbm.at[pl.ds(offt, GT2)], ir0.at[pl.ds(0, GT2)])
    pltpu.async_copy(A_my.at[ia0.at[pl.ds(0, GT2)]], av0.at[pl.ds(0, GT2)],
                     gsem0).wait()
    pltpu.async_copy(n_hbm.at[ir0.at[pl.ds(0, GT2)]], rv0.at[pl.ds(0, GT2)],
                     gsem0).wait()

    def trow(i, carry):
        for k in range(HV):
            sl = (i, pl.ds(k * 16, 16))
            ov0[sl] = av0[sl] - rv0[sl]
        return carry
    lax.fori_loop(0, GT2, trow, None)
    pltpu.sync_copy(ov0.at[pl.ds(0, GT2)], m_hbm.at[pl.ds(offt, GT2)])


_sc_scatter = functools.partial(
    pl.kernel,
    out_type=jax.ShapeDtypeStruct((NC, NA_PAD, H), jnp.float32),
    mesh=_sc_mesh(),
    scratch_types=[
        pltpu.VMEM((GC,), jnp.int32),
        pltpu.VMEM((GC,), jnp.int32),
        pltpu.VMEM((ST,), jnp.int32),
        pltpu.VMEM((GC, H), jnp.float32),
        pltpu.VMEM((GC, H), jnp.float32),
        pltpu.VMEM_SHARED((NA_PAD, H), jnp.float32),
        pltpu.SemaphoreType.DMA,
        pltpu.SemaphoreType.DMA,
    ],
)(_sc_scatter_body)


_sc_gather = functools.partial(
    pl.kernel,
    out_type=jax.ShapeDtypeStruct((N_BONDS, H), jnp.float32),
    mesh=_sc_mesh(),
    scratch_types=[
        pltpu.VMEM((GC2,), jnp.int32),
        pltpu.VMEM((GC2,), jnp.int32),
        pltpu.VMEM((GC2,), jnp.int32),
        pltpu.VMEM((GC2,), jnp.int32),
        pltpu.VMEM((GC2, H), jnp.float32),
        pltpu.VMEM((GC2, H), jnp.float32),
        pltpu.VMEM((GC2, H), jnp.float32),
        pltpu.VMEM((GC2, H), jnp.float32),
        pltpu.VMEM((GC2, H), jnp.float32),
        pltpu.VMEM((GC2, H), jnp.float32),
        pltpu.SemaphoreType.DMA,
        pltpu.SemaphoreType.DMA,
        pltpu.SemaphoreType.DMA,
        pltpu.SemaphoreType.DMA,
        pltpu.SemaphoreType.DMA,
        pltpu.SemaphoreType.DMA,
    ],
)(_sc_gather_body)


def _sc_readout_body(m_hbm, a2bf_hbm, r_hbm,
                     ix00, ix01, ix10, ix11, g00, g01, g10, g11, o0, o1,
                     isem0, isem1, gsem0, gsem1, osem0, osem1):
    cid = lax.axis_index("c")
    sid = lax.axis_index("s")
    wid = sid * NC + cid
    ix = ((ix00, ix01), (ix10, ix11))
    gv = ((g00, g01), (g10, g11))
    ov = (o0, o1)
    isem = (isem0, isem1)
    gsem = (gsem0, gsem1)
    osem = (osem0, osem1)

    def ri(c, b):
        off = (wid * ATOMS_PER_W + c * SC_ATOMS) * MAX_NB
        return ((a2bf_hbm.at[pl.ds(off, RG)], ix[b][0], isem[b]),
                (a2bf_hbm.at[pl.ds(off + RG, RG)], ix[b][1], isem[b]))

    def rg(b):
        return ((m_hbm.at[ix[b][0]], gv[b][0], gsem[b]),
                (m_hbm.at[ix[b][1]], gv[b][1], gsem[b]))

    def ro(c, b):
        aoff = wid * ATOMS_PER_W + c * SC_ATOMS
        return ((ov[b], r_hbm.at[pl.ds(aoff, SC_ATOMS)], osem[b]),)

    def start(ds):
        for s, d, sm in ds:
            pltpu.async_copy(s, d, sm)

    def wait(ds):
        for s, d, sm in ds:
            pltpu.make_async_copy(s, d, sm).wait()

    def compute(c, b):
        for half in range(2):
            g = gv[b][half]
            for a in range(SC_ATOMS // 2):
                acc = [g[a * MAX_NB, pl.ds(k * 16, 16)] for k in range(HV)]

                def nb(i, acc):
                    return tuple(acc[k] + g[a * MAX_NB + i, pl.ds(k * 16, 16)]
                                 for k in range(HV))
                acc = lax.fori_loop(1, MAX_NB, nb, tuple(acc))
                for k in range(HV):
                    ov[b][half * (SC_ATOMS // 2) + a,
                          pl.ds(k * 16, 16)] = acc[k]
        start(ro(c, b))

    start(ri(0, 0))
    start(ri(1, 1))
    wait(ri(0, 0))
    start(rg(0))

    wait(rg(0))
    wait(ri(1, 1))
    start(rg(1))
    compute(0, 0)
    start(ri(2, 0))

    wait(rg(1))
    wait(ri(2, 0))
    start(rg(0))
    compute(1, 1)
    start(ri(3, 1))

    def pair(jj, carry):
        c0 = 2 * jj
        wait(rg(0))
        wait(ri(c0 + 1, 1))
        start(rg(1))
        wait(ro(c0 - 2, 0))
        compute(c0, 0)
        start(ri(c0 + 2, 0))

        wait(rg(1))
        wait(ri(c0 + 2, 0))
        start(rg(0))
        wait(ro(c0 - 1, 1))
        compute(c0 + 1, 1)
        start(ri(c0 + 3, 1))
        return carry
    lax.fori_loop(1, R_CHUNKS // 2 - 1, pair, None)

    ct = R_CHUNKS - 2
    wait(rg(0))
    wait(ri(ct + 1, 1))
    start(rg(1))
    wait(ro(ct - 2, 0))
    compute(ct, 0)

    wait(rg(1))
    wait(ro(ct - 1, 1))
    compute(ct + 1, 1)
    wait(ro(ct, 0))
    wait(ro(ct + 1, 1))


_sc_readout = functools.partial(
    pl.kernel,
    out_type=jax.ShapeDtypeStruct((NA_PAD, H), jnp.float32),
    mesh=_sc_mesh(),
    scratch_types=[
        pltpu.VMEM((RG,), jnp.int32),
        pltpu.VMEM((RG,), jnp.int32),
        pltpu.VMEM((RG,), jnp.int32),
        pltpu.VMEM((RG,), jnp.int32),
        pltpu.VMEM((RG, H), jnp.float32),
        pltpu.VMEM((RG, H), jnp.float32),
        pltpu.VMEM((RG, H), jnp.float32),
        pltpu.VMEM((RG, H), jnp.float32),
        pltpu.VMEM((SC_ATOMS, H), jnp.float32),
        pltpu.VMEM((SC_ATOMS, H), jnp.float32),
        pltpu.SemaphoreType.DMA,
        pltpu.SemaphoreType.DMA,
        pltpu.SemaphoreType.DMA,
        pltpu.SemaphoreType.DMA,
        pltpu.SemaphoreType.DMA,
        pltpu.SemaphoreType.DMA,
    ],
)(_sc_readout_body)


BLK = 512
N_BLKS = N_BONDS // BLK


def _tc_init_body(fb_ref, wi_ref, wh_ref, inp_ref, n_ref):
    inp = jnp.dot(fb_ref[...], wi_ref[...], preferred_element_type=jnp.float32)
    inp_ref[...] = inp
    n_ref[...] = jnp.dot(jnp.maximum(inp, 0.0), wh_ref[...],
                         preferred_element_type=jnp.float32)


def _tc_init(f_bonds, W_i, W_h):
    return pl.pallas_call(
        _tc_init_body,
        grid=(N_BLKS,),
        in_specs=[
            pl.BlockSpec((BLK, BOND_FDIM), lambda i: (i, 0)),
            pl.BlockSpec((BOND_FDIM, H), lambda i: (0, 0)),
            pl.BlockSpec((H, H), lambda i: (0, 0)),
        ],
        out_specs=[
            pl.BlockSpec((BLK, H), lambda i: (i, 0)),
            pl.BlockSpec((BLK, H), lambda i: (i, 0)),
        ],
        out_shape=[
            jax.ShapeDtypeStruct((N_BONDS, H), jnp.float32),
            jax.ShapeDtypeStruct((N_BONDS, H), jnp.float32),
        ],
    )(f_bonds, W_i, W_h)


def _tc_mm_body(inp_ref, g_ref, wh_ref, n_ref):
    m = jnp.maximum(inp_ref[...] + g_ref[...], 0.0)
    n_ref[...] = jnp.dot(m, wh_ref[...], preferred_element_type=jnp.float32)


def _tc_mm(inp, g, W_h):
    return pl.pallas_call(
        _tc_mm_body,
        grid=(N_BLKS,),
        in_specs=[
            pl.BlockSpec((BLK, H), lambda i: (i, 0)),
            pl.BlockSpec((BLK, H), lambda i: (i, 0)),
            pl.BlockSpec((H, H), lambda i: (0, 0)),
        ],
        out_specs=pl.BlockSpec((BLK, H), lambda i: (i, 0)),
        out_shape=jax.ShapeDtypeStruct((N_BONDS, H), jnp.float32),
    )(inp, g, W_h)


def _tc_relu_body(inp_ref, g_ref, m_ref):
    m_ref[...] = jnp.maximum(inp_ref[...] + g_ref[...], 0.0)


def _tc_relu(inp, g):
    return pl.pallas_call(
        _tc_relu_body,
        grid=(N_BLKS,),
        in_specs=[
            pl.BlockSpec((BLK, H), lambda i: (i, 0)),
            pl.BlockSpec((BLK, H), lambda i: (i, 0)),
        ],
        out_specs=pl.BlockSpec((BLK, H), lambda i: (i, 0)),
        out_shape=jax.ShapeDtypeStruct((N_BONDS, H), jnp.float32),
    )(inp, g)


ABLK = 512
A_BLKS = NA_PAD // ABLK


def _tc_out_body(fa_ref, r_ref, wo_ref, bo_ref, scope_ref, mol_ref,
                 sums_ref, cnts_ref):
    i = pl.program_id(0)

    @pl.when(i == 0)
    def _():
        sums_ref[...] = jnp.zeros_like(sums_ref)
        cnts_ref[...] = jnp.zeros_like(cnts_ref)

    ah = jnp.dot(fa_ref[...], wo_ref[:ATOM_FDIM, :],
                 preferred_element_type=jnp.float32)
    ah = ah + jnp.dot(r_ref[...], wo_ref[ATOM_FDIM:, :],
                      preferred_element_type=jnp.float32)
    ah = jnp.maximum(ah + bo_ref[...], 0.0)

    seg = scope_ref[0, 0, :]
    mols = lax.broadcasted_iota(jnp.int32, (NM_PAD, ABLK), 0)
    onehot = (mols == seg[None, :]).astype(jnp.float32)
    sums_ref[...] += jnp.dot(onehot, ah, preferred_element_type=jnp.float32)
    cnts_ref[...] += jnp.dot(onehot, jnp.ones((ABLK, H), jnp.float32),
                             preferred_element_type=jnp.float32)

    @pl.when(i == A_BLKS - 1)
    def _():
        mol_ref[...] = sums_ref[...] / jnp.maximum(cnts_ref[...], 1.0)


def _tc_out(f_atoms_p, r, W_o, b_o, scope3):
    return pl.pallas_call(
        _tc_out_body,
        grid=(A_BLKS,),
        in_specs=[
            pl.BlockSpec((ABLK, ATOM_FDIM), lambda i: (i, 0)),
            pl.BlockSpec((ABLK, H), lambda i: (i, 0)),
            pl.BlockSpec((ATOM_FDIM + H, H), lambda i: (0, 0)),
            pl.BlockSpec((1, H), lambda i: (0, 0)),
            pl.BlockSpec((1, 1, ABLK), lambda i: (i, 0, 0)),
        ],
        out_specs=pl.BlockSpec((NM_PAD, H), lambda i: (0, 0)),
        out_shape=jax.ShapeDtypeStruct((NM_PAD, H), jnp.float32),
        scratch_shapes=[
            pltpu.VMEM((NM_PAD, H), jnp.float32),
            pltpu.VMEM((NM_PAD, H), jnp.float32),
        ],
    )(f_atoms_p, r, W_o, b_o, scope3)


def kernel(f_atoms, f_bonds, a2b, a2b2, b2a, b2revb, a_scope, b_scope,
           W_i, W_h, W_o, b_o):
    inp, n = _tc_init(f_bonds, W_i, W_h)
    m = None
    for t in range(DEPTH - 1):
        adump = _sc_scatter(n, a2b2)
        g = _sc_gather(n, b2a, b2revb, adump)
        if t < DEPTH - 2:
            n = _tc_mm(inp, g, W_h)
        else:
            m = _tc_relu(inp, g)

    a2bf = jnp.pad(a2b, ((0, NA_PAD - N_ATOMS), (0, 0))).reshape(-1)
    r = _sc_readout(m, a2bf)

    f_atoms_p = jnp.pad(f_atoms, ((0, NA_PAD - N_ATOMS), (0, 0)))
    scope_p = jnp.pad(a_scope, (0, NA_PAD - N_ATOMS))
    scope3 = scope_p.reshape(A_BLKS, 1, ABLK)
    mol = _tc_out(f_atoms_p, r, W_o, b_o.reshape(1, H), scope3)
    return mol[1:N_MOLS]


# speedup vs baseline: 1.0681x; 1.0681x over previous
import functools

import jax
import jax.numpy as jnp
from jax import lax
from jax.experimental import pallas as pl
from jax.experimental.pallas import tpu as pltpu
from jax.experimental.pallas import tpu_sc as plsc

N_ATOMS = 10000
N_BONDS = 320000
MAX_NB = 32
ATOM_FDIM = 128
BOND_FDIM = 144
HIDDEN = 128
DEPTH = 6
N_MOLS = 200

NA_PAD = 10240
NM_PAD = 256

NC, NS = 2, 16
NW = NC * NS
C = 40
BONDS_PER_W = N_BONDS // NW
G_CHUNKS = BONDS_PER_W // C
BONDS_PER_TILE = N_BONDS // NS
S_CHUNKS = BONDS_PER_TILE // C
AROWS_PER_TILE = NA_PAD // NS

ATOMS_PER_W = NA_PAD // NW
SC_ATOMS = 8
R_CHUNKS = ATOMS_PER_W // SC_ATOMS
RG = SC_ATOMS * MAX_NB // 2

H = HIDDEN
HV = H // 16


def _sc_mesh():
    return plsc.VectorSubcoreMesh(core_axis_name="c", subcore_axis_name="s")


GC = 120
GN = BONDS_PER_W // GC
GT = BONDS_PER_W - GN * GC
SN = BONDS_PER_TILE // GC
ST = BONDS_PER_TILE - SN * GC
ZN = AROWS_PER_TILE // GC
ZT = AROWS_PER_TILE - ZN * GC


def _sc_scatter_body(n_hbm, a2b2_hbm, adump_hbm,
                     six, six2, six_t, bv, bv2, A_sh, sem, sem2):
    cid = lax.axis_index("c")
    sid = lax.axis_index("s")
    wid = sid * NC + cid

    def zrow(i, carry):
        for k in range(HV):
            bv[i, pl.ds(k * 16, 16)] = jnp.zeros((16,), jnp.float32)
        return carry
    lax.fori_loop(0, GC, zrow, None)
    zbase = sid * AROWS_PER_TILE
    for j in range(ZN):
        pltpu.sync_copy(bv, A_sh.at[pl.ds(zbase + j * GC, GC)])
    pltpu.sync_copy(bv.at[pl.ds(0, ZT)],
                    A_sh.at[pl.ds(zbase + ZN * GC, ZT)])
    plsc.subcore_barrier()

    six_ = (six, six2)
    bv_ = (bv, bv2)
    sem_ = (sem, sem2)

    def s_f(c, b):
        off = sid * BONDS_PER_TILE + c * GC
        return ((a2b2_hbm.at[pl.ds(off, GC)], six_[b], sem_[b]),
                (n_hbm.at[pl.ds(off, GC)], bv_[b], sem_[b]))

    def s_start(ds):
        for s, d, sm in ds:
            pltpu.async_copy(s, d, sm)

    def s_wait(ds):
        for s, d, sm in ds:
            pltpu.make_async_copy(s, d, sm).wait()

    s_start(s_f(0, 0))

    def s_pair(jj, carry):
        c0 = 2 * jj
        s_wait(s_f(c0, 0))
        s_start(s_f(c0 + 1, 1))
        pltpu.sync_copy(bv_[0], A_sh.at[six_[0]], add=True)
        s_wait(s_f(c0 + 1, 1))
        s_start(s_f(c0 + 2, 0))
        pltpu.sync_copy(bv_[1], A_sh.at[six_[1]], add=True)
        return carry
    lax.fori_loop(0, SN // 2 - 1, s_pair, None)

    c0 = SN - 2
    s_wait(s_f(c0, 0))
    s_start(s_f(c0 + 1, 1))
    pltpu.sync_copy(bv_[0], A_sh.at[six_[0]], add=True)
    s_wait(s_f(c0 + 1, 1))
    pltpu.sync_copy(bv_[1], A_sh.at[six_[1]], add=True)

    offt = sid * BONDS_PER_TILE + SN * GC
    pltpu.sync_copy(a2b2_hbm.at[pl.ds(offt, ST)], six_t)
    pltpu.sync_copy(n_hbm.at[pl.ds(offt, ST)], bv.at[pl.ds(0, ST)])
    pltpu.sync_copy(bv.at[pl.ds(0, ST)], A_sh.at[six_t], add=True)
    plsc.subcore_barrier()

    zbase = sid * AROWS_PER_TILE
    pltpu.sync_copy(A_sh.at[pl.ds(zbase, AROWS_PER_TILE)],
                    adump_hbm.at[cid].at[pl.ds(zbase, AROWS_PER_TILE)])
    plsc.subcore_barrier()


GC2 = 104
GN2 = 96
GT2 = BONDS_PER_W - GN2 * GC2


def _sc_gather_body(n_hbm, inp_hbm, b2a_hbm, b2revb_hbm, adump_hbm, m_hbm,
                    ia0, ia1, ir0, ir1, av0, av1, rv0, rv1, bv0, bv1,
                    ov0, ov1, isem0, isem1, gsem0, gsem1, osem0, osem1):
    cid = lax.axis_index("c")
    sid = lax.axis_index("s")
    wid = sid * NC + cid
    ia = (ia0, ia1)
    ir = (ir0, ir1)
    av = (av0, av1)
    rv = (rv0, rv1)
    bv = (bv0, bv1)
    ov = (ov0, ov1)
    isem = (isem0, isem1)
    gsem = (gsem0, gsem1)
    osem = (osem0, osem1)
    A_my = adump_hbm.at[cid]

    def gi(c, b):
        off = wid * BONDS_PER_W + c * GC2
        return ((b2a_hbm.at[pl.ds(off, GC2)], ia[b], isem[b]),
                (b2revb_hbm.at[pl.ds(off, GC2)], ir[b], isem[b]))

    def gg(c, b):
        off = wid * BONDS_PER_W + c * GC2
        return ((A_my.at[ia[b]], av[b], gsem[b]),
                (n_hbm.at[ir[b]], rv[b], gsem[b]),
                (inp_hbm.at[pl.ds(off, GC2)], bv[b], gsem[b]))

    def go(c, b):
        off = wid * BONDS_PER_W + c * GC2
        return ((ov[b], m_hbm.at[pl.ds(off, GC2)], osem[b]),)

    def start(ds):
        for s, d, sm in ds:
            pltpu.async_copy(s, d, sm)

    def wait(ds):
        for s, d, sm in ds:
            pltpu.make_async_copy(s, d, sm).wait()

    def compute(c, b):
        def row(i, carry):
            for k in range(HV):
                sl = (i, pl.ds(k * 16, 16))
                ov[b][sl] = jnp.maximum(bv[b][sl] + av[b][sl] - rv[b][sl],
                                        0.0)
            return carry
        lax.fori_loop(0, GC2, row, None)
        start(go(c, b))

    start(gi(0, 0))
    start(gi(1, 1))
    wait(gi(0, 0))
    start(gg(0, 0))

    wait(gg(0, 0))
    wait(gi(1, 1))
    start(gg(1, 1))
    compute(0, 0)
    start(gi(2, 0))

    wait(gg(1, 1))
    wait(gi(2, 0))
    start(gg(2, 0))
    compute(1, 1)
    start(gi(3, 1))

    def pair(jj, carry):
        c0 = 2 * jj
        wait(gg(c0, 0))
        wait(gi(c0 + 1, 1))
        start(gg(c0 + 1, 1))
        wait(go(c0 - 2, 0))
        compute(c0, 0)
        start(gi(c0 + 2, 0))

        wait(gg(c0 + 1, 1))
        wait(gi(c0 + 2, 0))
        start(gg(c0 + 2, 0))
        wait(go(c0 - 1, 1))
        compute(c0 + 1, 1)
        start(gi(c0 + 3, 1))
        return carry
    lax.fori_loop(1, GN2 // 2 - 1, pair, None)

    ct = GN2 - 2
    wait(gg(ct, 0))
    wait(gi(ct + 1, 1))
    start(gg(ct + 1, 1))
    wait(go(ct - 2, 0))
    compute(ct, 0)

    wait(gg(ct + 1, 1))
    wait(go(ct - 1, 1))
    compute(ct + 1, 1)
    wait(go(ct, 0))
    wait(go(ct + 1, 1))

    offt = wid * BONDS_PER_W + GN2 * GC2
    pltpu.sync_copy(b2a_hbm.at[pl.ds(offt, GT2)], ia0.at[pl.ds(0, GT2)])
    pltpu.sync_copy(b2revb_hbm.at[pl.ds(offt, GT2)], ir0.at[pl.ds(0, GT2)])
    pltpu.async_copy(A_my.at[ia0.at[pl.ds(0, GT2)]], av0.at[pl.ds(0, GT2)],
                     gsem0).wait()
    pltpu.async_copy(n_hbm.at[ir0.at[pl.ds(0, GT2)]], rv0.at[pl.ds(0, GT2)],
                     gsem0).wait()
    pltpu.sync_copy(inp_hbm.at[pl.ds(offt, GT2)], bv0.at[pl.ds(0, GT2)])

    def trow(i, carry):
        for k in range(HV):
            sl = (i, pl.ds(k * 16, 16))
            ov0[sl] = jnp.maximum(bv0[sl] + av0[sl] - rv0[sl], 0.0)
        return carry
    lax.fori_loop(0, GT2, trow, None)
    pltpu.sync_copy(ov0.at[pl.ds(0, GT2)], m_hbm.at[pl.ds(offt, GT2)])


_sc_scatter = functools.partial(
    pl.kernel,
    out_type=jax.ShapeDtypeStruct((NC, NA_PAD, H), jnp.float32),
    mesh=_sc_mesh(),
    scratch_types=[
        pltpu.VMEM((GC,), jnp.int32),
        pltpu.VMEM((GC,), jnp.int32),
        pltpu.VMEM((ST,), jnp.int32),
        pltpu.VMEM((GC, H), jnp.float32),
        pltpu.VMEM((GC, H), jnp.float32),
        pltpu.VMEM_SHARED((NA_PAD, H), jnp.float32),
        pltpu.SemaphoreType.DMA,
        pltpu.SemaphoreType.DMA,
    ],
)(_sc_scatter_body)


_sc_gather = functools.partial(
    pl.kernel,
    out_type=jax.ShapeDtypeStruct((N_BONDS, H), jnp.float32),
    mesh=_sc_mesh(),
    scratch_types=[
        pltpu.VMEM((GC2,), jnp.int32),
        pltpu.VMEM((GC2,), jnp.int32),
        pltpu.VMEM((GC2,), jnp.int32),
        pltpu.VMEM((GC2,), jnp.int32),
        pltpu.VMEM((GC2, H), jnp.float32),
        pltpu.VMEM((GC2, H), jnp.float32),
        pltpu.VMEM((GC2, H), jnp.float32),
        pltpu.VMEM((GC2, H), jnp.float32),
        pltpu.VMEM((GC2, H), jnp.float32),
        pltpu.VMEM((GC2, H), jnp.float32),
        pltpu.VMEM((GC2, H), jnp.float32),
        pltpu.VMEM((GC2, H), jnp.float32),
        pltpu.SemaphoreType.DMA,
        pltpu.SemaphoreType.DMA,
        pltpu.SemaphoreType.DMA,
        pltpu.SemaphoreType.DMA,
        pltpu.SemaphoreType.DMA,
        pltpu.SemaphoreType.DMA,
    ],
)(_sc_gather_body)


def _sc_readout_body(m_hbm, a2bf_hbm, r_hbm,
                     ix00, ix01, ix10, ix11, g00, g01, g10, g11, o0, o1,
                     isem0, isem1, gsem0, gsem1, osem0, osem1):
    cid = lax.axis_index("c")
    sid = lax.axis_index("s")
    wid = sid * NC + cid
    ix = ((ix00, ix01), (ix10, ix11))
    gv = ((g00, g01), (g10, g11))
    ov = (o0, o1)
    isem = (isem0, isem1)
    gsem = (gsem0, gsem1)
    osem = (osem0, osem1)

    def ri(c, b):
        off = (wid * ATOMS_PER_W + c * SC_ATOMS) * MAX_NB
        return ((a2bf_hbm.at[pl.ds(off, RG)], ix[b][0], isem[b]),
                (a2bf_hbm.at[pl.ds(off + RG, RG)], ix[b][1], isem[b]))

    def rg(b):
        return ((m_hbm.at[ix[b][0]], gv[b][0], gsem[b]),
                (m_hbm.at[ix[b][1]], gv[b][1], gsem[b]))

    def ro(c, b):
        aoff = wid * ATOMS_PER_W + c * SC_ATOMS
        return ((ov[b], r_hbm.at[pl.ds(aoff, SC_ATOMS)], osem[b]),)

    def start(ds):
        for s, d, sm in ds:
            pltpu.async_copy(s, d, sm)

    def wait(ds):
        for s, d, sm in ds:
            pltpu.make_async_copy(s, d, sm).wait()

    def compute(c, b):
        for half in range(2):
            g = gv[b][half]
            for a in range(SC_ATOMS // 2):
                acc = [g[a * MAX_NB, pl.ds(k * 16, 16)] for k in range(HV)]

                def nb(i, acc):
                    return tuple(acc[k] + g[a * MAX_NB + i, pl.ds(k * 16, 16)]
                                 for k in range(HV))
                acc = lax.fori_loop(1, MAX_NB, nb, tuple(acc))
                for k in range(HV):
                    ov[b][half * (SC_ATOMS // 2) + a,
                          pl.ds(k * 16, 16)] = acc[k]
        start(ro(c, b))

    start(ri(0, 0))
    start(ri(1, 1))
    wait(ri(0, 0))
    start(rg(0))

    wait(rg(0))
    wait(ri(1, 1))
    start(rg(1))
    compute(0, 0)
    start(ri(2, 0))

    wait(rg(1))
    wait(ri(2, 0))
    start(rg(0))
    compute(1, 1)
    start(ri(3, 1))

    def pair(jj, carry):
        c0 = 2 * jj
        wait(rg(0))
        wait(ri(c0 + 1, 1))
        start(rg(1))
        wait(ro(c0 - 2, 0))
        compute(c0, 0)
        start(ri(c0 + 2, 0))

        wait(rg(1))
        wait(ri(c0 + 2, 0))
        start(rg(0))
        wait(ro(c0 - 1, 1))
        compute(c0 + 1, 1)
        start(ri(c0 + 3, 1))
        return carry
    lax.fori_loop(1, R_CHUNKS // 2 - 1, pair, None)

    ct = R_CHUNKS - 2
    wait(rg(0))
    wait(ri(ct + 1, 1))
    start(rg(1))
    wait(ro(ct - 2, 0))
    compute(ct, 0)

    wait(rg(1))
    wait(ro(ct - 1, 1))
    compute(ct + 1, 1)
    wait(ro(ct, 0))
    wait(ro(ct + 1, 1))


_sc_readout = functools.partial(
    pl.kernel,
    out_type=jax.ShapeDtypeStruct((NA_PAD, H), jnp.float32),
    mesh=_sc_mesh(),
    scratch_types=[
        pltpu.VMEM((RG,), jnp.int32),
        pltpu.VMEM((RG,), jnp.int32),
        pltpu.VMEM((RG,), jnp.int32),
        pltpu.VMEM((RG,), jnp.int32),
        pltpu.VMEM((RG, H), jnp.float32),
        pltpu.VMEM((RG, H), jnp.float32),
        pltpu.VMEM((RG, H), jnp.float32),
        pltpu.VMEM((RG, H), jnp.float32),
        pltpu.VMEM((SC_ATOMS, H), jnp.float32),
        pltpu.VMEM((SC_ATOMS, H), jnp.float32),
        pltpu.SemaphoreType.DMA,
        pltpu.SemaphoreType.DMA,
        pltpu.SemaphoreType.DMA,
        pltpu.SemaphoreType.DMA,
        pltpu.SemaphoreType.DMA,
        pltpu.SemaphoreType.DMA,
    ],
)(_sc_readout_body)


BLK = 512
N_BLKS = N_BONDS // BLK


def _tc_init_body(fb_ref, wi_ref, wh_ref, inp_ref, n_ref):
    inp = jnp.dot(fb_ref[...], wi_ref[...], preferred_element_type=jnp.float32)
    inp_ref[...] = inp
    n_ref[...] = jnp.dot(jnp.maximum(inp, 0.0), wh_ref[...],
                         preferred_element_type=jnp.float32)


def _tc_init(f_bonds, W_i, W_h):
    return pl.pallas_call(
        _tc_init_body,
        grid=(N_BLKS,),
        in_specs=[
            pl.BlockSpec((BLK, BOND_FDIM), lambda i: (i, 0)),
            pl.BlockSpec((BOND_FDIM, H), lambda i: (0, 0)),
            pl.BlockSpec((H, H), lambda i: (0, 0)),
        ],
        out_specs=[
            pl.BlockSpec((BLK, H), lambda i: (i, 0)),
            pl.BlockSpec((BLK, H), lambda i: (i, 0)),
        ],
        out_shape=[
            jax.ShapeDtypeStruct((N_BONDS, H), jnp.float32),
            jax.ShapeDtypeStruct((N_BONDS, H), jnp.float32),
        ],
    )(f_bonds, W_i, W_h)


def _tc_mm_body(m_ref, wh_ref, n_ref):
    n_ref[...] = jnp.dot(m_ref[...], wh_ref[...],
                         preferred_element_type=jnp.float32)


def _tc_mm(m, W_h):
    return pl.pallas_call(
        _tc_mm_body,
        grid=(N_BLKS,),
        in_specs=[
            pl.BlockSpec((BLK, H), lambda i: (i, 0)),
            pl.BlockSpec((H, H), lambda i: (0, 0)),
        ],
        out_specs=pl.BlockSpec((BLK, H), lambda i: (i, 0)),
        out_shape=jax.ShapeDtypeStruct((N_BONDS, H), jnp.float32),
    )(m, W_h)


ABLK = 512
A_BLKS = NA_PAD // ABLK


def _tc_out_body(fa_ref, r_ref, wo_ref, bo_ref, scope_ref, mol_ref,
                 sums_ref, cnts_ref):
    i = pl.program_id(0)

    @pl.when(i == 0)
    def _():
        sums_ref[...] = jnp.zeros_like(sums_ref)
        cnts_ref[...] = jnp.zeros_like(cnts_ref)

    ah = jnp.dot(fa_ref[...], wo_ref[:ATOM_FDIM, :],
                 preferred_element_type=jnp.float32)
    ah = ah + jnp.dot(r_ref[...], wo_ref[ATOM_FDIM:, :],
                      preferred_element_type=jnp.float32)
    ah = jnp.maximum(ah + bo_ref[...], 0.0)

    seg = scope_ref[0, 0, :]
    mols = lax.broadcasted_iota(jnp.int32, (NM_PAD, ABLK), 0)
    onehot = (mols == seg[None, :]).astype(jnp.float32)
    sums_ref[...] += jnp.dot(onehot, ah, preferred_element_type=jnp.float32)
    cnts_ref[...] += jnp.dot(onehot, jnp.ones((ABLK, H), jnp.float32),
                             preferred_element_type=jnp.float32)

    @pl.when(i == A_BLKS - 1)
    def _():
        mol_ref[...] = sums_ref[...] / jnp.maximum(cnts_ref[...], 1.0)


def _tc_out(f_atoms_p, r, W_o, b_o, scope3):
    return pl.pallas_call(
        _tc_out_body,
        grid=(A_BLKS,),
        in_specs=[
            pl.BlockSpec((ABLK, ATOM_FDIM), lambda i: (i, 0)),
            pl.BlockSpec((ABLK, H), lambda i: (i, 0)),
            pl.BlockSpec((ATOM_FDIM + H, H), lambda i: (0, 0)),
            pl.BlockSpec((1, H), lambda i: (0, 0)),
            pl.BlockSpec((1, 1, ABLK), lambda i: (i, 0, 0)),
        ],
        out_specs=pl.BlockSpec((NM_PAD, H), lambda i: (0, 0)),
        out_shape=jax.ShapeDtypeStruct((NM_PAD, H), jnp.float32),
        scratch_shapes=[
            pltpu.VMEM((NM_PAD, H), jnp.float32),
            pltpu.VMEM((NM_PAD, H), jnp.float32),
        ],
    )(f_atoms_p, r, W_o, b_o, scope3)


def kernel(f_atoms, f_bonds, a2b, a2b2, b2a, b2revb, a_scope, b_scope,
           W_i, W_h, W_o, b_o):
    inp, n = _tc_init(f_bonds, W_i, W_h)
    m = None
    for t in range(DEPTH - 1):
        adump = _sc_scatter(n, a2b2)
        m = _sc_gather(n, inp, b2a, b2revb, adump)
        if t < DEPTH - 2:
            n = _tc_mm(m, W_h)

    a2bf = jnp.pad(a2b, ((0, NA_PAD - N_ATOMS), (0, 0))).reshape(-1)
    r = _sc_readout(m, a2bf)

    f_atoms_p = jnp.pad(f_atoms, ((0, NA_PAD - N_ATOMS), (0, 0)))
    scope_p = jnp.pad(a_scope, (0, NA_PAD - N_ATOMS))
    scope3 = scope_p.reshape(A_BLKS, 1, ABLK)
    mol = _tc_out(f_atoms_p, r, W_o, b_o.reshape(1, H), scope3)
    return mol[1:N_MOLS]
